# baseline (speedup 1.0000x reference)
import functools

import jax
import jax.numpy as jnp
from jax import lax
from jax.experimental import pallas as pl
from jax.experimental.pallas import tpu as pltpu
from jax.experimental.pallas import tpu_sc as plsc

N = 10000
E = 320000
NP = 10112
RPT = 632
CH = 128
E_PAD = 327680
NCHUNK = E_PAD // CH
CPT_F = NCHUNK // 16
CPT_E = NCHUNK // 32
DEG_W = 16
IB = 16
NB_F = CPT_F // IB
NB_E = CPT_E // IB

_f32 = jnp.float32
_MESH = plsc.VectorSubcoreMesh(core_axis_name="c", subcore_axis_name="s")


def _deg_body(src2, dst2, zeros, onesw, degout, degin, sidx, ones_v, acc, sem):
    c = lax.axis_index("c")
    s = lax.axis_index("s")
    rbase = pl.multiple_of(s * RPT, 8)
    pltpu.sync_copy(zeros.at[pl.ds(rbase, RPT)], acc.at[pl.ds(rbase, RPT)])
    pltpu.sync_copy(onesw, ones_v)
    plsc.subcore_barrier()

    def outer(b, carry):
        cbase = s * CPT_F + b * IB

        @pl.when(c == 0)
        def _():
            pltpu.sync_copy(src2.at[pl.ds(cbase, IB)], sidx)

        @pl.when(c == 1)
        def _():
            pltpu.sync_copy(dst2.at[pl.ds(cbase, IB)], sidx)

        def step(i, carry2):
            pltpu.sync_copy(ones_v, acc.at[sidx.at[i]], add=True)
            return carry2

        return lax.fori_loop(0, IB, step, carry)

    lax.fori_loop(0, NB_F, outer, 0)
    plsc.subcore_barrier()

    @pl.when(c == 0)
    def _():
        pltpu.sync_copy(acc.at[pl.ds(rbase, RPT)], degout.at[pl.ds(rbase, RPT)])

    @pl.when(c == 1)
    def _():
        pltpu.sync_copy(acc.at[pl.ds(rbase, RPT)], degin.at[pl.ds(rbase, RPT)])


_deg_call = pl.kernel(
    _deg_body,
    out_type=[jax.ShapeDtypeStruct((NP, 128), _f32)] * 2,
    mesh=_MESH,
    scratch_types=[
        pltpu.VMEM((IB, CH), jnp.int32),
        pltpu.VMEM((CH, 128), _f32),
        pltpu.VMEM_SHARED((NP, 128), _f32),
        pltpu.SemaphoreType.DMA,
    ],
)


CH_P = 64
NBUF = 4
NCHUNK_P = E_PAD // CH_P
CPT_PF = NCHUNK_P // 16
CPT_PE = NCHUNK_P // 32


def _make_prop_body(cpt, edge_split):
    lead = NBUF - 1

    def body(src2, dst2, tabA, tabB, zeros, outA, outB,
             sw0, sw1, dw, rb0, rb1, rb2, rb3, acc, g0, g1, g2, g3):
        c = lax.axis_index("c")
        s = lax.axis_index("s")
        rbase = pl.multiple_of(s * RPT, 8)
        pltpu.sync_copy(zeros.at[pl.ds(rbase, RPT)], acc.at[pl.ds(rbase, RPT)])
        tb = (c * 16 + s) * cpt if edge_split else s * cpt
        plsc.subcore_barrier()
        swins = (sw0, sw1)
        rbufs = (rb0, rb1, rb2, rb3)
        gsems = (g0, g1, g2, g3)

        def gather_to(idx, b):
            if edge_split:
                pltpu.async_copy(tabA.at[idx], rbufs[b], gsems[b])
            else:
                @pl.when(c == 0)
                def _():
                    pltpu.async_copy(tabA.at[idx], rbufs[b], gsems[b])

                @pl.when(c == 1)
                def _():
                    pltpu.async_copy(tabB.at[idx], rbufs[b], gsems[b])

        pltpu.sync_copy(src2.at[pl.ds(pl.multiple_of(tb, 8), IB)], sw0)
        for g in range(lead):
            gather_to(sw0.at[g], g % NBUF)

        def step(i, carry):
            g = i + lead

            @pl.when(g < cpt)
            def _():
                @pl.when(g % IB == 0)
                def _():
                    for p in range(2):
                        @pl.when((g // IB) % 2 == p)
                        def _():
                            pltpu.sync_copy(
                                src2.at[pl.ds(pl.multiple_of(tb + g, 8), IB)],
                                swins[p])

                row = g % IB
                for p in range(2):
                    for b in range(NBUF):
                        @pl.when(jnp.logical_and((g // IB) % 2 == p,
                                                 g % NBUF == b))
                        def _():
                            gather_to(swins[p].at[row], b)

            @pl.when(i % IB == 0)
            def _():
                pltpu.sync_copy(
                    dst2.at[pl.ds(pl.multiple_of(tb + i, 8), IB)], dw)

            row = i % IB
            for b in range(NBUF):
                @pl.when(i % NBUF == b)
                def _():
                    pltpu.make_async_copy(tabA.at[pl.ds(0, CH_P)], rbufs[b],
                                          gsems[b]).wait()
                    pltpu.sync_copy(rbufs[b], acc.at[dw.at[row]], add=True)
            return carry

        lax.fori_loop(0, cpt, step, 0)
        plsc.subcore_barrier()

        @pl.when(c == 0)
        def _():
            pltpu.sync_copy(acc.at[pl.ds(rbase, RPT)],
                            outA.at[pl.ds(rbase, RPT)])

        @pl.when(c == 1)
        def _():
            pltpu.sync_copy(acc.at[pl.ds(rbase, RPT)],
                            outB.at[pl.ds(rbase, RPT)])

    return body


_PROP_SCRATCH = [
    pltpu.VMEM((IB, CH_P), jnp.int32),
    pltpu.VMEM((IB, CH_P), jnp.int32),
    pltpu.VMEM((IB, CH_P), jnp.int32),
    pltpu.VMEM((CH_P, 128), _f32),
    pltpu.VMEM((CH_P, 128), _f32),
    pltpu.VMEM((CH_P, 128), _f32),
    pltpu.VMEM((CH_P, 128), _f32),
    pltpu.VMEM_SHARED((NP, 128), _f32),
    pltpu.SemaphoreType.DMA,
    pltpu.SemaphoreType.DMA,
    pltpu.SemaphoreType.DMA,
    pltpu.SemaphoreType.DMA,
]

_prop_edge_call2 = pl.kernel(
    _make_prop_body(CPT_PE, True),
    out_type=[jax.ShapeDtypeStruct((NP, 128), _f32)] * 2,
    mesh=_MESH,
    scratch_types=_PROP_SCRATCH,
)

_prop_feat_call = pl.kernel(
    _make_prop_body(CPT_PF, False),
    out_type=[jax.ShapeDtypeStruct((NP, 128), _f32)] * 2,
    mesh=_MESH,
    scratch_types=_PROP_SCRATCH,
)


def _prop_edge_call(src2, dst2, tab, zeros):
    return _prop_edge_call2(src2, dst2, tab, tab, zeros)


def _norm(deg):
    return lax.rsqrt(jnp.maximum(deg, 1.0))


def _prep_body(x_ref, dego_ref, xs_ref):
    xs_ref[...] = x_ref[...] * _norm(dego_ref[...])


def _layer1_body(p0_ref, p1_ref, degi_ref, dego_ref, w_ref, b_ref,
                 ha_ref, hb_ref):
    agg = (p0_ref[...] + p1_ref[...]) * _norm(degi_ref[...])
    h = jnp.dot(agg, w_ref[...], preferred_element_type=_f32) + b_ref[...]
    h = jnp.maximum(h, 0.0) * _norm(dego_ref[...])
    ha_ref[...] = h[:, :128]
    hb_ref[...] = h[:, 128:]


def _layer2_body(aa_ref, ab_ref, degi_ref, dego_ref, w_ref, b_ref,
                 ha_ref, hb_ref):
    nd = _norm(degi_ref[...])
    w = w_ref[...]
    h = (jnp.dot(aa_ref[...] * nd, w[:128], preferred_element_type=_f32)
         + jnp.dot(ab_ref[...] * nd, w[128:], preferred_element_type=_f32)
         + b_ref[...])
    h = jnp.maximum(h, 0.0) * _norm(dego_ref[...])
    ha_ref[...] = h[:, :128]
    hb_ref[...] = h[:, 128:]


def _final_body(aa_ref, ab_ref, degi_ref, w3_ref, b3_ref, wl_ref, bl_ref,
                h_ref, y_ref):
    nd = _norm(degi_ref[...])
    w3 = w3_ref[...]
    h = (jnp.dot(aa_ref[...] * nd, w3[:128], preferred_element_type=_f32)
         + jnp.dot(ab_ref[...] * nd, w3[128:], preferred_element_type=_f32)
         + b3_ref[...])
    h = jnp.maximum(h, 0.0)
    h_ref[...] = h
    y_ref[...] = jnp.dot(h, wl_ref[...], preferred_element_type=_f32) + bl_ref[...]


def _rows(r, c):
    return pl.BlockSpec((r, c), lambda i: (i, 0))


def _whole(shape):
    return pl.BlockSpec(shape, lambda i: (0, 0))


_prep_call = pl.pallas_call(
    _prep_body,
    grid=(16,),
    in_specs=[_rows(RPT, 128), _rows(RPT, 1)],
    out_specs=_rows(RPT, 128),
    out_shape=jax.ShapeDtypeStruct((NP, 128), _f32),
)

_layer1_call = pl.pallas_call(
    _layer1_body,
    grid=(16,),
    in_specs=[_rows(RPT, 128), _rows(RPT, 128), _rows(RPT, 1), _rows(RPT, 1),
              _whole((128, 256)), _whole((1, 256))],
    out_specs=[_rows(RPT, 128), _rows(RPT, 128)],
    out_shape=[jax.ShapeDtypeStruct((NP, 128), _f32)] * 2,
)

_layer2_call = pl.pallas_call(
    _layer2_body,
    grid=(16,),
    in_specs=[_rows(RPT, 128), _rows(RPT, 128), _rows(RPT, 1), _rows(RPT, 1),
              _whole((256, 256)), _whole((1, 256))],
    out_specs=[_rows(RPT, 128), _rows(RPT, 128)],
    out_shape=[jax.ShapeDtypeStruct((NP, 128), _f32)] * 2,
)

_final_call = pl.pallas_call(
    _final_body,
    grid=(25,),
    in_specs=[_rows(400, 128), _rows(400, 128), _rows(400, 1),
              _whole((256, 256)), _whole((1, 256)),
              _whole((256, 128)), _whole((1, 128))],
    out_specs=[_rows(400, 256), _rows(400, 128)],
    out_shape=[jax.ShapeDtypeStruct((N, 256), _f32),
               jax.ShapeDtypeStruct((N, 128), _f32)],
)


def _deg_jnp(src, dst):
    ones = jnp.ones((E,), _f32)
    do = jax.ops.segment_sum(ones, src, num_segments=N)
    di = jax.ops.segment_sum(ones, dst, num_segments=N)
    pad = jnp.zeros((NP - N,), _f32)
    return (jnp.broadcast_to(jnp.concatenate([do, pad])[:, None], (NP, DEG_W)),
            jnp.broadcast_to(jnp.concatenate([di, pad])[:, None], (NP, DEG_W)))


def _prop_jnp(src, dst, tab):
    agg = jax.ops.segment_sum(jnp.take(tab[:N], src, axis=0), dst,
                              num_segments=N)
    return jnp.concatenate([agg, jnp.zeros((NP - N, 128), _f32)], axis=0)


def kernel(in_feat, edge_index, W1, b1, W2, b2, W3, b3, Wl, bl):
    src = edge_index[0]
    dst = edge_index[1]
    padi = jnp.full((E_PAD - E,), N, jnp.int32)
    src2 = jnp.concatenate([src, padi]).reshape(NCHUNK, CH)
    dst2 = jnp.concatenate([dst, padi]).reshape(NCHUNK, CH)
    x_p = jnp.concatenate(
        [in_feat, jnp.zeros((NP - N, in_feat.shape[1]), _f32)], axis=0)
    onesw = jnp.ones((CH, 128), _f32)
    z128 = jnp.zeros((NP, 128), _f32)

    src64 = src2.reshape(NCHUNK_P, CH_P)
    dst64 = dst2.reshape(NCHUNK_P, CH_P)

    degow, degiw = _deg_call(src2, dst2, z128, onesw)
    dego2 = degow[:, 0:1]
    degi2 = degiw[:, 0:1]

    xs = _prep_call(x_p, dego2)
    p0, p1 = _prop_edge_call(src64, dst64, xs, z128)
    h1a, h1b = _layer1_call(p0, p1, degi2, dego2, W1, b1.reshape(1, -1))

    a2a, a2b = _prop_feat_call(src64, dst64, h1a, h1b, z128)
    h2a, h2b = _layer2_call(a2a, a2b, degi2, dego2, W2, b2.reshape(1, -1))

    a3a, a3b = _prop_feat_call(src64, dst64, h2a, h2b, z128)
    wlp = jnp.concatenate([Wl, jnp.zeros((256, 128 - 40), _f32)], axis=1)
    blp = jnp.concatenate([bl, jnp.zeros((128 - 40,), _f32)]).reshape(1, -1)
    h, yp = _final_call(a3a, a3b, degi2, W3, b3.reshape(1, -1), wlp, blp)
    return (h, yp[:, :40])


# speedup vs baseline: 1.0114x; 1.0114x over previous
import jax
import jax.numpy as jnp
from jax import lax
from jax.experimental import pallas as pl
from jax.experimental.pallas import tpu as pltpu
from jax.experimental.pallas import tpu_sc as plsc

N = 10000
E = 320000
NP = 10112
RPT = 632
CH = 128
E_PAD = 327680
NCHUNK = E_PAD // CH
CPT_F = NCHUNK // 16
CPT_E = NCHUNK // 32
IB = 16
NB_F = CPT_F // IB

_f32 = jnp.float32
_MESH = plsc.VectorSubcoreMesh(core_axis_name="c", subcore_axis_name="s")


def _deg_body(src2, dst2, zeros, onesw, degout, degin, sidx, ones_v, acc, sem):
    c = lax.axis_index("c")
    s = lax.axis_index("s")
    rbase = pl.multiple_of(s * RPT, 8)
    pltpu.sync_copy(zeros.at[pl.ds(rbase, RPT)], acc.at[pl.ds(rbase, RPT)])
    pltpu.sync_copy(onesw, ones_v)
    plsc.subcore_barrier()

    def outer(b, carry):
        cbase = s * CPT_F + b * IB

        @pl.when(c == 0)
        def _():
            pltpu.sync_copy(src2.at[pl.ds(cbase, IB)], sidx)

        @pl.when(c == 1)
        def _():
            pltpu.sync_copy(dst2.at[pl.ds(cbase, IB)], sidx)

        def step(i, carry2):
            pltpu.sync_copy(ones_v, acc.at[sidx.at[i]], add=True)
            return carry2

        return lax.fori_loop(0, IB, step, carry)

    lax.fori_loop(0, NB_F, outer, 0)
    plsc.subcore_barrier()

    @pl.when(c == 0)
    def _():
        pltpu.sync_copy(acc.at[pl.ds(rbase, RPT)], degout.at[pl.ds(rbase, RPT)])

    @pl.when(c == 1)
    def _():
        pltpu.sync_copy(acc.at[pl.ds(rbase, RPT)], degin.at[pl.ds(rbase, RPT)])


_deg_call = pl.kernel(
    _deg_body,
    out_type=[jax.ShapeDtypeStruct((NP, 128), _f32)] * 2,
    mesh=_MESH,
    scratch_types=[
        pltpu.VMEM((IB, CH), jnp.int32),
        pltpu.VMEM((CH, 128), _f32),
        pltpu.VMEM_SHARED((NP, 128), _f32),
        pltpu.SemaphoreType.DMA,
    ],
)


def _make_prop_body(cpt, edge_split):
    def body(src2, dst2, tabA, tabB, zeros, outA, outB,
             sw0, sw1, dw, r0, r1, acc, gs0, gs1):
        c = lax.axis_index("c")
        s = lax.axis_index("s")
        rbase = pl.multiple_of(s * RPT, 8)
        pltpu.sync_copy(zeros.at[pl.ds(rbase, RPT)], acc.at[pl.ds(rbase, RPT)])
        tb = (c * 16 + s) * cpt if edge_split else s * cpt
        plsc.subcore_barrier()
        swins = (sw0, sw1)
        rbufs = (r0, r1)
        gsems = (gs0, gs1)

        def gather_to(idx, b):
            if edge_split:
                pltpu.async_copy(tabA.at[idx], rbufs[b], gsems[b])
            else:
                @pl.when(c == 0)
                def _():
                    pltpu.async_copy(tabA.at[idx], rbufs[b], gsems[b])

                @pl.when(c == 1)
                def _():
                    pltpu.async_copy(tabB.at[idx], rbufs[b], gsems[b])

        pltpu.sync_copy(src2.at[pl.ds(pl.multiple_of(tb, 8), IB)], sw0)
        gather_to(sw0.at[0], 0)

        def step(i, carry):
            g = i + 1

            @pl.when(g < cpt)
            def _():
                @pl.when(g % IB == 0)
                def _():
                    for p in range(2):
                        @pl.when((g // IB) % 2 == p)
                        def _():
                            pltpu.sync_copy(
                                src2.at[pl.ds(pl.multiple_of(tb + g, 8), IB)],
                                swins[p])

                row = g % IB
                for p in range(2):
                    for b in range(2):
                        @pl.when(jnp.logical_and((g // IB) % 2 == p,
                                                 g % 2 == b))
                        def _():
                            gather_to(swins[p].at[row], b)

            @pl.when(i % IB == 0)
            def _():
                pltpu.sync_copy(
                    dst2.at[pl.ds(pl.multiple_of(tb + i, 8), IB)], dw)

            row = i % IB
            for b in range(2):
                @pl.when(i % 2 == b)
                def _():
                    pltpu.make_async_copy(tabA.at[pl.ds(0, CH)], rbufs[b],
                                          gsems[b]).wait()
                    pltpu.sync_copy(rbufs[b], acc.at[dw.at[row]], add=True)
            return carry

        lax.fori_loop(0, cpt, step, 0)
        plsc.subcore_barrier()

        @pl.when(c == 0)
        def _():
            pltpu.sync_copy(acc.at[pl.ds(rbase, RPT)],
                            outA.at[pl.ds(rbase, RPT)])

        @pl.when(c == 1)
        def _():
            pltpu.sync_copy(acc.at[pl.ds(rbase, RPT)],
                            outB.at[pl.ds(rbase, RPT)])

    return body


_PROP_SCRATCH = [
    pltpu.VMEM((IB, CH), jnp.int32),
    pltpu.VMEM((IB, CH), jnp.int32),
    pltpu.VMEM((IB, CH), jnp.int32),
    pltpu.VMEM((CH, 128), _f32),
    pltpu.VMEM((CH, 128), _f32),
    pltpu.VMEM_SHARED((NP, 128), _f32),
    pltpu.SemaphoreType.DMA,
    pltpu.SemaphoreType.DMA,
]

_prop_edge_call2 = pl.kernel(
    _make_prop_body(CPT_E, True),
    out_type=[jax.ShapeDtypeStruct((NP, 128), _f32)] * 2,
    mesh=_MESH,
    scratch_types=_PROP_SCRATCH,
)

_prop_feat_call = pl.kernel(
    _make_prop_body(CPT_F, False),
    out_type=[jax.ShapeDtypeStruct((NP, 128), _f32)] * 2,
    mesh=_MESH,
    scratch_types=_PROP_SCRATCH,
)


def _prop_edge_call(src2, dst2, tab, zeros):
    return _prop_edge_call2(src2, dst2, tab, tab, zeros)


def _norm(deg):
    return lax.rsqrt(jnp.maximum(deg, 1.0))


def _prep_body(x_ref, dego_ref, xs_ref):
    xs_ref[...] = x_ref[...] * _norm(dego_ref[...])


def _layer1_body(p0_ref, p1_ref, degi_ref, dego_ref, w_ref, b_ref,
                 ha_ref, hb_ref):
    agg = (p0_ref[...] + p1_ref[...]) * _norm(degi_ref[...])
    h = jnp.dot(agg, w_ref[...], preferred_element_type=_f32) + b_ref[...]
    h = jnp.maximum(h, 0.0) * _norm(dego_ref[...])
    ha_ref[...] = h[:, :128]
    hb_ref[...] = h[:, 128:]


def _layer2_body(aa_ref, ab_ref, degi_ref, dego_ref, w_ref, b_ref,
                 ha_ref, hb_ref):
    nd = _norm(degi_ref[...])
    w = w_ref[...]
    h = (jnp.dot(aa_ref[...] * nd, w[:128], preferred_element_type=_f32)
         + jnp.dot(ab_ref[...] * nd, w[128:], preferred_element_type=_f32)
         + b_ref[...])
    h = jnp.maximum(h, 0.0) * _norm(dego_ref[...])
    ha_ref[...] = h[:, :128]
    hb_ref[...] = h[:, 128:]


def _final_body(aa_ref, ab_ref, degi_ref, w3_ref, b3_ref, wl_ref, bl_ref,
                h_ref, y_ref):
    nd = _norm(degi_ref[...])
    w3 = w3_ref[...]
    h = (jnp.dot(aa_ref[...] * nd, w3[:128], preferred_element_type=_f32)
         + jnp.dot(ab_ref[...] * nd, w3[128:], preferred_element_type=_f32)
         + b3_ref[...])
    h = jnp.maximum(h, 0.0)
    h_ref[...] = h
    y_ref[...] = jnp.dot(h, wl_ref[...], preferred_element_type=_f32) + bl_ref[...]


def _rows(r, c):
    return pl.BlockSpec((r, c), lambda i: (i, 0))


def _whole(shape):
    return pl.BlockSpec(shape, lambda i: (0, 0))


_prep_call = pl.pallas_call(
    _prep_body,
    grid=(16,),
    in_specs=[_rows(RPT, 128), _rows(RPT, 1)],
    out_specs=_rows(RPT, 128),
    out_shape=jax.ShapeDtypeStruct((NP, 128), _f32),
)

_layer1_call = pl.pallas_call(
    _layer1_body,
    grid=(16,),
    in_specs=[_rows(RPT, 128), _rows(RPT, 128), _rows(RPT, 1), _rows(RPT, 1),
              _whole((128, 256)), _whole((1, 256))],
    out_specs=[_rows(RPT, 128), _rows(RPT, 128)],
    out_shape=[jax.ShapeDtypeStruct((NP, 128), _f32)] * 2,
)

_layer2_call = pl.pallas_call(
    _layer2_body,
    grid=(16,),
    in_specs=[_rows(RPT, 128), _rows(RPT, 128), _rows(RPT, 1), _rows(RPT, 1),
              _whole((256, 256)), _whole((1, 256))],
    out_specs=[_rows(RPT, 128), _rows(RPT, 128)],
    out_shape=[jax.ShapeDtypeStruct((NP, 128), _f32)] * 2,
)

_final_call = pl.pallas_call(
    _final_body,
    grid=(25,),
    in_specs=[_rows(400, 128), _rows(400, 128), _rows(400, 1),
              _whole((256, 256)), _whole((1, 256)),
              _whole((256, 128)), _whole((1, 128))],
    out_specs=[_rows(400, 256), _rows(400, 128)],
    out_shape=[jax.ShapeDtypeStruct((N, 256), _f32),
               jax.ShapeDtypeStruct((N, 128), _f32)],
)


def kernel(in_feat, edge_index, W1, b1, W2, b2, W3, b3, Wl, bl):
    src = edge_index[0]
    dst = edge_index[1]
    padi = jnp.full((E_PAD - E,), N, jnp.int32)
    src2 = jnp.concatenate([src, padi]).reshape(NCHUNK, CH)
    dst2 = jnp.concatenate([dst, padi]).reshape(NCHUNK, CH)
    x_p = jnp.concatenate(
        [in_feat, jnp.zeros((NP - N, in_feat.shape[1]), _f32)], axis=0)
    onesw = jnp.ones((CH, 128), _f32)
    z128 = jnp.zeros((NP, 128), _f32)

    degow, degiw = _deg_call(src2, dst2, z128, onesw)
    dego2 = degow[:, 0:1]
    degi2 = degiw[:, 0:1]

    xs = _prep_call(x_p, dego2)
    p0, p1 = _prop_edge_call(src2, dst2, xs, z128)
    h1a, h1b = _layer1_call(p0, p1, degi2, dego2, W1, b1.reshape(1, -1))

    a2a, a2b = _prop_feat_call(src2, dst2, h1a, h1b, z128)
    h2a, h2b = _layer2_call(a2a, a2b, degi2, dego2, W2, b2.reshape(1, -1))

    a3a, a3b = _prop_feat_call(src2, dst2, h2a, h2b, z128)
    wlp = jnp.concatenate([Wl, jnp.zeros((256, 128 - 40), _f32)], axis=1)
    blp = jnp.concatenate([bl, jnp.zeros((128 - 40,), _f32)]).reshape(1, -1)
    h, yp = _final_call(a3a, a3b, degi2, W3, b3.reshape(1, -1), wlp, blp)
    return (h, yp[:, :40])
